# Initial kernel scaffold; baseline (speedup 1.0000x reference)
import jax
import jax.numpy as jnp
from jax.experimental import pallas as pl


def kernel(pos_ids, table):
    raise NotImplementedError("write your pallas kernel here")


# speedup vs baseline: 7.1234x; 7.1234x over previous
import functools

import jax
import jax.numpy as jnp
from jax import lax
from jax.experimental import pallas as pl
from jax.experimental.pallas import tpu as pltpu
from jax.experimental.pallas import tpu_sc as plsc

_B = 4096 * 200
_D = 50
_V = 1000
_CHUNK = 128
_NW = 32
_B_PER_W = _B // _NW
_N_CHUNK = _B_PER_W // _CHUNK
_SLAB = 200


def _emb_body(idx_hbm, table_hbm, out_hbm, table_sp, table_tv, idx_v, rows_v, gsem):
    sid = lax.axis_index("s")
    cid = lax.axis_index("c")
    wid = sid * 2 + cid
    base = wid * _B_PER_W

    @pl.when(sid == 0)
    def _():
        def stage(k, carry):
            r = k * _SLAB
            pltpu.sync_copy(table_hbm.at[pl.ds(r, _SLAB)], table_tv)
            pltpu.sync_copy(table_tv, table_sp.at[pl.ds(r, _SLAB)])
            return carry

        lax.fori_loop(0, _V // _SLAB, stage, 0)

    pltpu.sync_copy(idx_hbm.at[pl.ds(base, _B_PER_W)], idx_v)
    plsc.subcore_barrier()

    def body(c, carry):
        off = c * _CHUNK
        pltpu.async_copy(
            table_sp.at[idx_v.at[pl.ds(off, _CHUNK)]], rows_v, gsem
        ).wait()
        pltpu.sync_copy(rows_v, out_hbm.at[pl.ds(base + off, _CHUNK)])
        return carry

    lax.fori_loop(0, _N_CHUNK, body, 0)


def kernel(pos_ids, table):
    idx = pos_ids.reshape(_B)
    mesh = plsc.VectorSubcoreMesh(core_axis_name="c", subcore_axis_name="s")
    run = pl.kernel(
        _emb_body,
        mesh=mesh,
        out_type=jax.ShapeDtypeStruct((_B, _D), jnp.float32),
        scratch_types=[
            pltpu.VMEM_SHARED((_V, _D), jnp.float32),
            pltpu.VMEM((_SLAB, _D), jnp.float32),
            pltpu.VMEM((_B_PER_W,), jnp.int32),
            pltpu.VMEM((_CHUNK, _D), jnp.float32),
            pltpu.SemaphoreType.DMA,
        ],
    )
    out = run(idx, table)
    return out.reshape(4096, 200, _D)


# speedup vs baseline: 8.5960x; 1.2067x over previous
import functools

import jax
import jax.numpy as jnp
from jax import lax
from jax.experimental import pallas as pl
from jax.experimental.pallas import tpu as pltpu
from jax.experimental.pallas import tpu_sc as plsc

_B = 4096 * 200
_D = 50
_V = 1000
_CHUNK = 128
_GPS = 2
_SUPER = _CHUNK * _GPS
_NW = 32
_B_PER_W = _B // _NW
_N_SUPER = _B_PER_W // _SUPER
_SLAB = 200


def _emb_body(idx_hbm, table_hbm, out_hbm, table_sp, table_tv, idx_v,
              rows_a, rows_b, gsem, osem_a, osem_b):
    sid = lax.axis_index("s")
    cid = lax.axis_index("c")
    wid = sid * 2 + cid
    base = wid * _B_PER_W

    @pl.when(sid == 0)
    def _():
        def stage(k, carry):
            r = k * _SLAB
            pltpu.sync_copy(table_hbm.at[pl.ds(r, _SLAB)], table_tv)
            pltpu.sync_copy(table_tv, table_sp.at[pl.ds(r, _SLAB)])
            return carry

        lax.fori_loop(0, _V // _SLAB, stage, 0)

    pltpu.sync_copy(idx_hbm.at[pl.ds(base, _B_PER_W)], idx_v)
    plsc.subcore_barrier()

    def fill(s, rows):
        for j in range(_GPS):
            off = s * _SUPER + j * _CHUNK
            pltpu.async_copy(
                table_sp.at[idx_v.at[pl.ds(off, _CHUNK)]],
                rows.at[pl.ds(j * _CHUNK, _CHUNK)],
                gsem,
            ).wait()

    def body(g, carry):
        for b, rows, osem in ((0, rows_a, osem_a), (1, rows_b, osem_b)):
            s = g * 2 + b

            @pl.when(s >= 2)
            def _():
                pltpu.make_async_copy(
                    rows,
                    out_hbm.at[pl.ds(base + (s - 2) * _SUPER, _SUPER)],
                    osem,
                ).wait()

            fill(s, rows)
            pltpu.async_copy(
                rows, out_hbm.at[pl.ds(base + s * _SUPER, _SUPER)], osem
            )
        return carry

    lax.fori_loop(0, _N_SUPER // 2, body, 0)

    for rows, osem, s in ((rows_a, osem_a, _N_SUPER - 2),
                          (rows_b, osem_b, _N_SUPER - 1)):
        pltpu.make_async_copy(
            rows, out_hbm.at[pl.ds(base + s * _SUPER, _SUPER)], osem
        ).wait()


def kernel(pos_ids, table):
    idx = pos_ids.reshape(_B)
    mesh = plsc.VectorSubcoreMesh(core_axis_name="c", subcore_axis_name="s")
    run = pl.kernel(
        _emb_body,
        mesh=mesh,
        out_type=jax.ShapeDtypeStruct((_B, _D), jnp.float32),
        scratch_types=[
            pltpu.VMEM_SHARED((_V, _D), jnp.float32),
            pltpu.VMEM((_SLAB, _D), jnp.float32),
            pltpu.VMEM((_B_PER_W,), jnp.int32),
            pltpu.VMEM((_SUPER, _D), jnp.float32),
            pltpu.VMEM((_SUPER, _D), jnp.float32),
            pltpu.SemaphoreType.DMA,
            pltpu.SemaphoreType.DMA,
            pltpu.SemaphoreType.DMA,
        ],
    )
    out = run(idx, table)
    return out.reshape(4096, 200, _D)
